# baseline (device time: 205627 ns/iter reference)
import jax
import jax.numpy as jnp
from jax import lax
from jax.experimental import pallas as pl
from jax.experimental.pallas import tpu as pltpu

N_DEV = 8
E_LOCAL = 4
N_TOK = 2048
D_IN = 512
H_OUT = 1024
R = N_TOK // N_DEV
N_HOP = N_DEV - 1


def kernel(x, router_W, route_idx, expert_W):
    del router_W

    def body(x_ref, idx_ref, w_ref, out_ref, comm_ref,
             rs_send, rs_recv, ag_send, ag_recv):
        d = lax.axis_index("i")
        left = (d - 1 + N_DEV) % N_DEV
        right = (d + 1) % N_DEV

        barrier_sem = pltpu.get_barrier_semaphore()
        for nbr in (left, right):
            pl.semaphore_signal(
                barrier_sem, inc=1,
                device_id=(nbr,), device_id_type=pl.DeviceIdType.MESH,
            )
        pl.semaphore_wait(barrier_sem, 2)

        route = idx_ref[:, :]
        acc = jnp.zeros((N_TOK, H_OUT), jnp.float32)
        for j in range(E_LOCAL):
            e = d * E_LOCAL + j
            xm = jnp.where(route == e, x_ref[:, :], 0.0)
            acc = acc + jnp.dot(xm, w_ref[j],
                                preferred_element_type=jnp.float32)
        out_ref[:, :] = acc

        for h in range(N_HOP):
            s = (d - h + 2 * N_DEV) % N_DEV
            r = (d - h - 1 + 2 * N_DEV) % N_DEV
            rdma = pltpu.make_async_remote_copy(
                src_ref=out_ref.at[pl.ds(s * R, R), :],
                dst_ref=comm_ref.at[h],
                send_sem=rs_send.at[h],
                recv_sem=rs_recv.at[h],
                device_id=(right,),
                device_id_type=pl.DeviceIdType.MESH,
            )
            rdma.start()
            rdma.wait()
            out_ref[pl.ds(r * R, R), :] += comm_ref[h]

        for h in range(N_HOP):
            g = (d + 1 - h + 2 * N_DEV) % N_DEV
            rdma = pltpu.make_async_remote_copy(
                src_ref=out_ref.at[pl.ds(g * R, R), :],
                dst_ref=out_ref.at[pl.ds(g * R, R), :],
                send_sem=ag_send.at[h],
                recv_sem=ag_recv.at[h],
                device_id=(right,),
                device_id_type=pl.DeviceIdType.MESH,
            )
            rdma.start()
            rdma.wait()

    return pl.pallas_call(
        body,
        out_shape=jax.ShapeDtypeStruct((N_TOK, H_OUT), jnp.float32),
        in_specs=[
            pl.BlockSpec(memory_space=pltpu.VMEM),
            pl.BlockSpec(memory_space=pltpu.VMEM),
            pl.BlockSpec(memory_space=pltpu.VMEM),
        ],
        out_specs=pl.BlockSpec(memory_space=pltpu.VMEM),
        scratch_shapes=[
            pltpu.VMEM((N_HOP, R, H_OUT), jnp.float32),
            pltpu.SemaphoreType.DMA((N_HOP,)),
            pltpu.SemaphoreType.DMA((N_HOP,)),
            pltpu.SemaphoreType.DMA((N_HOP,)),
            pltpu.SemaphoreType.DMA((N_HOP,)),
        ],
        compiler_params=pltpu.CompilerParams(collective_id=0),
    )(x, route_idx, expert_W)


# device time: 94985 ns/iter; 2.1648x vs baseline; 2.1648x over previous
import jax
import jax.numpy as jnp
from jax import lax
from jax.experimental import pallas as pl
from jax.experimental.pallas import tpu as pltpu

N_DEV = 8
E_LOCAL = 4
N_TOK = 2048
D_IN = 512
H_OUT = 1024

X_AXIS = (1, lambda d: (d ^ (d >> 1)) & 1)
Y_AXIS = (3, lambda d: (d >> 1) & 1)
Z_AXIS = (4, lambda d: (d >> 2) & 1)

FLOWS = (
    (0, 384, (X_AXIS, Y_AXIS, Z_AXIS)),
    (384, 384, (Y_AXIS, Z_AXIS, X_AXIS)),
    (768, 256, (Z_AXIS, X_AXIS, Y_AXIS)),
)

HALVES = (1024, 512, 256)
STAGE_OFF = (0, 1024, 1536)


def kernel(x, router_W, route_idx, expert_W):
    del router_W

    def body(x_ref, idx_ref, w_ref, out_ref,
             comm_a, comm_b, comm_c, send_sems, rs_recv, ag_recv):
        comm = (comm_a, comm_b, comm_c)
        d = lax.axis_index("i")
        route = idx_ref[:, :]

        barrier_sem = pltpu.get_barrier_semaphore()
        for m, _ in (X_AXIS, Y_AXIS, Z_AXIS):
            pl.semaphore_signal(
                barrier_sem, inc=1,
                device_id=(d ^ m,), device_id_type=pl.DeviceIdType.MESH,
            )
        pl.semaphore_wait(barrier_sem, 3)

        def rs_start(fi, k, off):
            c0, w, order = FLOWS[fi]
            m, s = order[k]
            sv = s(d)
            half = HALVES[k]
            send_off = off + (1 - sv) * half
            desc = pltpu.make_async_remote_copy(
                src_ref=out_ref.at[pl.ds(send_off, half), pl.ds(c0, w)],
                dst_ref=comm[fi].at[pl.ds(STAGE_OFF[k], half), :],
                send_sem=send_sems.at[fi],
                recv_sem=rs_recv.at[fi, k],
                device_id=(d ^ m,),
                device_id_type=pl.DeviceIdType.MESH,
            )
            desc.start()
            return desc, off + sv * half

        offs = [0, 0, 0]
        descs = [None, None, None]

        for fi in range(3):
            c0, w, _ = FLOWS[fi]
            acc = jnp.zeros((N_TOK, w), jnp.float32)
            for j in range(E_LOCAL):
                e = d * E_LOCAL + j
                xm = jnp.where(route == e, x_ref[:, :], 0.0)
                acc = acc + jnp.dot(xm, w_ref[j, :, c0:c0 + w],
                                    preferred_element_type=jnp.float32)
            out_ref[:, pl.ds(c0, w)] = acc
            descs[fi], offs[fi] = rs_start(fi, 0, 0)

        for k in range(3):
            for fi in range(3):
                c0, w, _ = FLOWS[fi]
                descs[fi].wait()
                half = HALVES[k]
                out_ref[pl.ds(offs[fi], half), pl.ds(c0, w)] += (
                    comm[fi][pl.ds(STAGE_OFF[k], half), :]
                )
                if k < 2:
                    descs[fi], offs[fi] = rs_start(fi, k + 1, offs[fi])

        for k in range(3):
            cur = HALVES[2 - k]
            for fi in range(3):
                c0, w, order = FLOWS[fi]
                m, s = order[2 - k]
                desc = pltpu.make_async_remote_copy(
                    src_ref=out_ref.at[pl.ds(offs[fi], cur), pl.ds(c0, w)],
                    dst_ref=out_ref.at[pl.ds(offs[fi], cur), pl.ds(c0, w)],
                    send_sem=send_sems.at[fi],
                    recv_sem=ag_recv.at[fi, k],
                    device_id=(d ^ m,),
                    device_id_type=pl.DeviceIdType.MESH,
                )
                desc.start()
                descs[fi] = desc
                offs[fi] = offs[fi] - s(d) * cur
            for fi in range(3):
                descs[fi].wait()

    return pl.pallas_call(
        body,
        out_shape=jax.ShapeDtypeStruct((N_TOK, H_OUT), jnp.float32),
        in_specs=[
            pl.BlockSpec(memory_space=pltpu.VMEM),
            pl.BlockSpec(memory_space=pltpu.VMEM),
            pl.BlockSpec(memory_space=pltpu.VMEM),
        ],
        out_specs=pl.BlockSpec(memory_space=pltpu.VMEM),
        scratch_shapes=[
            pltpu.VMEM((1792, 384), jnp.float32),
            pltpu.VMEM((1792, 384), jnp.float32),
            pltpu.VMEM((1792, 256), jnp.float32),
            pltpu.SemaphoreType.DMA((3,)),
            pltpu.SemaphoreType.DMA((3, 3)),
            pltpu.SemaphoreType.DMA((3, 3)),
        ],
        compiler_params=pltpu.CompilerParams(collective_id=0),
    )(x, route_idx, expert_W)


# device time: 56675 ns/iter; 3.6282x vs baseline; 1.6760x over previous
import jax
import jax.numpy as jnp
from jax import lax
from jax.experimental import pallas as pl
from jax.experimental.pallas import tpu as pltpu

N_DEV = 8
E_LOCAL = 4
N_TOK = 2048
D_IN = 512
H_OUT = 1024

X_AXIS = (1, lambda d: (d ^ (d >> 1)) & 1)
Y_AXIS = (3, lambda d: (d >> 1) & 1)
Z_AXIS = (4, lambda d: (d >> 2) & 1)

FLOWS = (
    (0, 384, (X_AXIS, Y_AXIS, Z_AXIS)),
    (384, 384, (Y_AXIS, Z_AXIS, X_AXIS)),
    (768, 256, (Z_AXIS, X_AXIS, Y_AXIS)),
)

HALVES = (1024, 512, 256)
STAGE_OFF = (0, 1024, 1536)
BF = jnp.bfloat16


def kernel(x, router_W, route_idx, expert_W):
    del router_W

    def body(x_ref, idx_ref, w_ref, out_ref, red_ref,
             comm_a, comm_b, comm_c, send_sems, rs_recv, ag_recv):
        comm = (comm_a, comm_b, comm_c)
        d = lax.axis_index("i")

        barrier_sem = pltpu.get_barrier_semaphore()
        for m, _ in (X_AXIS, Y_AXIS, Z_AXIS):
            pl.semaphore_signal(
                barrier_sem, inc=1,
                device_id=(d ^ m,), device_id_type=pl.DeviceIdType.MESH,
            )
        pl.semaphore_wait(barrier_sem, 3)

        def partial_rows(fi, row_off, nrows):
            c0, w, _ = FLOWS[fi]
            rs = pl.ds(row_off, nrows)
            xs = x_ref[rs, :].astype(BF)
            rt = idx_ref[rs, :]
            acc = jnp.zeros((nrows, w), jnp.float32)
            for j in range(E_LOCAL):
                xm = jnp.where(rt == d * E_LOCAL + j, xs, 0)
                acc = acc + jnp.dot(xm, w_ref[j, :, c0:c0 + w].astype(BF),
                                    preferred_element_type=jnp.float32)
            red_ref[rs, pl.ds(c0, w)] = acc.astype(BF)

        def exchange(fi, src_off, nrows, axis, recv_sems, slot, into_comm):
            c0, w, _ = FLOWS[fi]
            m, _ = axis
            src = red_ref.at[pl.ds(src_off, nrows), pl.ds(c0, w)]
            if into_comm:
                dst = comm[fi].at[pl.ds(STAGE_OFF[slot], nrows), :]
            else:
                dst = src
            desc = pltpu.make_async_remote_copy(
                src_ref=src, dst_ref=dst,
                send_sem=send_sems.at[fi],
                recv_sem=recv_sems.at[fi, slot],
                device_id=(d ^ m,),
                device_id_type=pl.DeviceIdType.MESH,
            )
            desc.start()
            return desc

        offs = [0, 0, 0]
        descs = [None, None, None]

        for fi in range(3):
            _, s = FLOWS[fi][2][0]
            sv = s(d)
            partial_rows(fi, (1 - sv) * 1024, 1024)
            descs[fi] = exchange(fi, (1 - sv) * 1024, 1024, FLOWS[fi][2][0],
                                 rs_recv, 0, True)
            offs[fi] = sv * 1024
        for fi in range(3):
            partial_rows(fi, offs[fi], 1024)

        for fi in range(3):
            c0, w, order = FLOWS[fi]
            descs[fi].wait()
            _, s2 = order[1]
            sv2 = s2(d)
            so = offs[fi] + (1 - sv2) * 512
            ko = offs[fi] + sv2 * 512
            red_ref[pl.ds(so, 512), pl.ds(c0, w)] += (
                comm[fi][pl.ds(STAGE_OFF[0] + (so - offs[fi]), 512), :]
            )
            descs[fi] = exchange(fi, so, 512, order[1], rs_recv, 1, True)
            red_ref[pl.ds(ko, 512), pl.ds(c0, w)] += (
                comm[fi][pl.ds(STAGE_OFF[0] + (ko - offs[fi]), 512), :]
            )
            offs[fi] = ko

        for fi in range(3):
            c0, w, order = FLOWS[fi]
            descs[fi].wait()
            red_ref[pl.ds(offs[fi], 512), pl.ds(c0, w)] += (
                comm[fi][pl.ds(STAGE_OFF[1], 512), :]
            )
            descs[fi] = exchange(fi, offs[fi], 512, order[2],
                                 rs_recv, 2, True)

        ag_offs = [0, 0, 0]
        for fi in range(3):
            c0, w, order = FLOWS[fi]
            descs[fi].wait()
            red_ref[pl.ds(offs[fi], 512), pl.ds(c0, w)] += (
                comm[fi][pl.ds(STAGE_OFF[2], 512), :]
            )
            _, s1 = order[1]
            descs[fi] = exchange(fi, offs[fi], 512, order[1],
                                 ag_recv, 1, False)
            ag_offs[fi] = offs[fi] - s1(d) * 512

        for fi in range(3):
            c0, w, order = FLOWS[fi]
            descs[fi].wait()
            descs[fi] = exchange(fi, ag_offs[fi], 1024, order[0],
                                 ag_recv, 2, False)
            out_ref[pl.ds(ag_offs[fi], 1024), pl.ds(c0, w)] = (
                red_ref[pl.ds(ag_offs[fi], 1024), pl.ds(c0, w)]
                .astype(jnp.float32)
            )

        for fi in range(3):
            c0, w, order = FLOWS[fi]
            descs[fi].wait()
            _, s = order[0]
            rcv = (1 - s(d)) * 1024
            out_ref[pl.ds(rcv, 1024), pl.ds(c0, w)] = (
                red_ref[pl.ds(rcv, 1024), pl.ds(c0, w)].astype(jnp.float32)
            )

    return pl.pallas_call(
        body,
        out_shape=jax.ShapeDtypeStruct((N_TOK, H_OUT), jnp.float32),
        in_specs=[
            pl.BlockSpec(memory_space=pltpu.VMEM),
            pl.BlockSpec(memory_space=pltpu.VMEM),
            pl.BlockSpec(memory_space=pltpu.VMEM),
        ],
        out_specs=pl.BlockSpec(memory_space=pltpu.VMEM),
        scratch_shapes=[
            pltpu.VMEM((N_TOK, H_OUT), BF),
            pltpu.VMEM((2048, 384), BF),
            pltpu.VMEM((2048, 384), BF),
            pltpu.VMEM((2048, 256), BF),
            pltpu.SemaphoreType.DMA((3,)),
            pltpu.SemaphoreType.DMA((3, 3)),
            pltpu.SemaphoreType.DMA((3, 3)),
        ],
        compiler_params=pltpu.CompilerParams(collective_id=0),
    )(x, route_idx, expert_W)
